# baseline (speedup 1.0000x reference)
import jax
import jax.numpy as jnp
from jax import lax
from jax.experimental import pallas as pl
from jax.experimental.pallas import tpu as pltpu
from jax.experimental.pallas import tpu_sc as plsc

_L, _B, _D = 200, 4096, 64
_BW = 128
_CL = 2
_NCH = _L // _CL


def _transpose_chunk(gbuf, tbuf):
    lane = lax.iota(jnp.int32, 16)
    for j in range(_CL):
        src = gbuf.at[j]

        def body(c, carry):
            cidx = jnp.zeros((16,), jnp.int32) + c
            for g in range(_BW // 16):
                v = plsc.load_gather(src, [g * 16 + lane, cidx])
                tbuf[j, c, pl.ds(g * 16, 16)] = v
            return carry

        lax.fori_loop(0, _D, body, 0)


def _emb_body(table_hbm, idx_hbm, out_hbm,
              idx_v, ga, gb, ta, tb, gsa, gsb, ssa, ssb):
    wid = lax.axis_index("s") * 2 + lax.axis_index("c")
    b0 = wid * _BW
    pltpu.sync_copy(idx_hbm.at[:, pl.ds(b0, _BW)], idx_v)

    def gstart(c, buf, sem):
        for j in range(_CL):
            pltpu.async_copy(
                table_hbm.at[idx_v.at[c * _CL + j]], buf.at[j], sem)

    def gwait(c, buf, sem):
        for j in range(_CL):
            pltpu.make_async_copy(
                table_hbm.at[idx_v.at[c * _CL + j]], buf.at[j], sem
            ).wait()

    def sstart(c, buf, sem):
        pltpu.async_copy(
            buf, out_hbm.at[pl.ds(c * _CL, _CL), :, pl.ds(b0, _BW)], sem)

    def swait(c, buf, sem):
        pltpu.make_async_copy(
            buf, out_hbm.at[pl.ds(c * _CL, _CL), :, pl.ds(b0, _BW)], sem
        ).wait()

    gstart(0, ga, gsa)

    def step(j, carry):
        c0 = 2 * j
        c1 = c0 + 1
        gstart(c1, gb, gsb)
        gwait(c0, ga, gsa)

        @pl.when(j > 0)
        def _():
            swait(c0 - 2, ta, ssa)

        _transpose_chunk(ga, ta)
        sstart(c0, ta, ssa)

        @pl.when(c0 + 2 < _NCH)
        def _():
            gstart(c0 + 2, ga, gsa)

        gwait(c1, gb, gsb)

        @pl.when(j > 0)
        def _():
            swait(c1 - 2, tb, ssb)

        _transpose_chunk(gb, tb)
        sstart(c1, tb, ssb)
        return carry

    lax.fori_loop(0, _NCH // 2, step, 0)
    swait(_NCH - 2, ta, ssa)
    swait(_NCH - 1, tb, ssb)


def kernel(indices, labels, table):
    idx = indices.astype(jnp.int32)
    table128 = jnp.pad(table, ((0, 0), (0, 128 - _D)))
    mesh = plsc.VectorSubcoreMesh(core_axis_name="c", subcore_axis_name="s")
    out = pl.kernel(
        _emb_body,
        mesh=mesh,
        compiler_params=pltpu.CompilerParams(
            use_tc_tiling_on_sc=True, needs_layout_passes=False),
        out_type=jax.ShapeDtypeStruct((_L, _D, _B), jnp.float32),
        scratch_types=[
            pltpu.VMEM((_L, _BW), jnp.int32),
            pltpu.VMEM((_CL, _BW, 128), jnp.float32),
            pltpu.VMEM((_CL, _BW, 128), jnp.float32),
            pltpu.VMEM((_CL, _D, _BW), jnp.float32),
            pltpu.VMEM((_CL, _D, _BW), jnp.float32),
            pltpu.SemaphoreType.DMA,
            pltpu.SemaphoreType.DMA,
            pltpu.SemaphoreType.DMA,
            pltpu.SemaphoreType.DMA,
        ],
    )(table128, idx)
    return (jnp.transpose(out, (0, 2, 1)), labels)


# speedup vs baseline: 1.2175x; 1.2175x over previous
import jax
import jax.numpy as jnp
from jax import lax
from jax.experimental import pallas as pl
from jax.experimental.pallas import tpu as pltpu
from jax.experimental.pallas import tpu_sc as plsc

_L, _B, _D = 200, 4096, 64
_BW = 128
_CL = 2
_NCH = _L // _CL


def _transpose_chunk(gbuf, tbuf):
    lane = lax.iota(jnp.int32, 16)
    bvecs = [g * 16 + lane for g in range(_BW // 16)]
    zero = jnp.zeros((16,), jnp.int32)

    def body(ci, carry):
        work = []
        for dc in range(2):
            c = 2 * ci + dc
            cidx = zero + c
            for j in range(_CL):
                for g in range(_BW // 16):
                    v = plsc.load_gather(gbuf.at[j], [bvecs[g], cidx])
                    work.append((j, c, g, v))
        for j, c, g, v in work:
            tbuf[j, c, pl.ds(g * 16, 16)] = v
        return carry

    lax.fori_loop(0, _D // 2, body, 0)


def _emb_body(table_hbm, idx_hbm, out_hbm,
              idx_v, ga, gb, ta, tb, gsa, gsb, ssa, ssb):
    wid = lax.axis_index("s") * 2 + lax.axis_index("c")
    b0 = wid * _BW
    pltpu.sync_copy(idx_hbm.at[:, pl.ds(b0, _BW)], idx_v)

    def gstart(c, buf, sem):
        for j in range(_CL):
            pltpu.async_copy(
                table_hbm.at[idx_v.at[c * _CL + j]], buf.at[j], sem)

    def gwait(c, buf, sem):
        for j in range(_CL):
            pltpu.make_async_copy(
                table_hbm.at[idx_v.at[c * _CL + j]], buf.at[j], sem
            ).wait()

    def sstart(c, buf, sem):
        pltpu.async_copy(
            buf, out_hbm.at[pl.ds(c * _CL, _CL), :, pl.ds(b0, _BW)], sem)

    def swait(c, buf, sem):
        pltpu.make_async_copy(
            buf, out_hbm.at[pl.ds(c * _CL, _CL), :, pl.ds(b0, _BW)], sem
        ).wait()

    gstart(0, ga, gsa)

    def step(j, carry):
        c0 = 2 * j
        c1 = c0 + 1
        gstart(c1, gb, gsb)
        gwait(c0, ga, gsa)

        @pl.when(j > 0)
        def _():
            swait(c0 - 2, ta, ssa)

        _transpose_chunk(ga, ta)
        sstart(c0, ta, ssa)

        @pl.when(c0 + 2 < _NCH)
        def _():
            gstart(c0 + 2, ga, gsa)

        gwait(c1, gb, gsb)

        @pl.when(j > 0)
        def _():
            swait(c1 - 2, tb, ssb)

        _transpose_chunk(gb, tb)
        sstart(c1, tb, ssb)
        return carry

    lax.fori_loop(0, _NCH // 2, step, 0)
    swait(_NCH - 2, ta, ssa)
    swait(_NCH - 1, tb, ssb)


def kernel(indices, labels, table):
    idx = indices.astype(jnp.int32)
    table128 = jnp.pad(table, ((0, 0), (0, 128 - _D)))
    mesh = plsc.VectorSubcoreMesh(core_axis_name="c", subcore_axis_name="s")
    out = pl.kernel(
        _emb_body,
        mesh=mesh,
        compiler_params=pltpu.CompilerParams(
            use_tc_tiling_on_sc=True, needs_layout_passes=False),
        out_type=jax.ShapeDtypeStruct((_L, _D, _B), jnp.float32),
        scratch_types=[
            pltpu.VMEM((_L, _BW), jnp.int32),
            pltpu.VMEM((_CL, _BW, 128), jnp.float32),
            pltpu.VMEM((_CL, _BW, 128), jnp.float32),
            pltpu.VMEM((_CL, _D, _BW), jnp.float32),
            pltpu.VMEM((_CL, _D, _BW), jnp.float32),
            pltpu.SemaphoreType.DMA,
            pltpu.SemaphoreType.DMA,
            pltpu.SemaphoreType.DMA,
            pltpu.SemaphoreType.DMA,
        ],
    )(table128, idx)
    return (jnp.transpose(out, (0, 2, 1)), labels)
